# Initial kernel scaffold; baseline (speedup 1.0000x reference)
import jax
import jax.numpy as jnp
from jax.experimental import pallas as pl


def kernel(x):
    raise NotImplementedError("write your pallas kernel here")


# speedup vs baseline: 3.5869x; 3.5869x over previous
import functools

import jax
import jax.numpy as jnp
from jax import lax
from jax.experimental import pallas as pl
from jax.experimental.pallas import tpu as pltpu
from jax.experimental.pallas import tpu_sc as plsc

N_ROWS = 16384
N_COLS = 256
N_OUT = 16
L = 16

NC = 2
NS = 16
NW = NC * NS
ROWS_PER_W = N_ROWS // NW
CHUNK = 128
NCHUNK = ROWS_PER_W // CHUNK


_mesh = plsc.VectorSubcoreMesh(core_axis_name="c", subcore_axis_name="s")


@functools.partial(
    pl.kernel,
    mesh=_mesh,
    out_type=jax.ShapeDtypeStruct((N_ROWS, N_OUT), jnp.float32),
    scratch_types=[
        pltpu.VMEM((CHUNK, N_COLS), jnp.float32),
        pltpu.VMEM((CHUNK, N_OUT), jnp.float32),
    ],
)
def _pool_sc(x_hbm, out_hbm, in_v, out_v):
    wid = lax.axis_index("s") * NC + lax.axis_index("c")
    base = wid * ROWS_PER_W

    def chunk_body(ci, _):
        row0 = base + ci * CHUNK
        pltpu.sync_copy(x_hbm.at[pl.ds(row0, CHUNK)], in_v)

        def row_body(r, _):
            v = in_v[r, pl.ds(0, L)]
            for k in range(1, 16):
                v = jnp.maximum(v, in_v[r, pl.ds(k * L, L)])
            out_v[r, :] = v
            return 0

        lax.fori_loop(0, CHUNK, row_body, 0)
        pltpu.sync_copy(out_v, out_hbm.at[pl.ds(row0, CHUNK)])
        return 0

    lax.fori_loop(0, NCHUNK, chunk_body, 0)


def kernel(x):
    return _pool_sc(x)


# speedup vs baseline: 3.9404x; 1.0985x over previous
import functools

import jax
import jax.numpy as jnp
from jax import lax
from jax.experimental import pallas as pl
from jax.experimental.pallas import tpu as pltpu
from jax.experimental.pallas import tpu_sc as plsc

N_ROWS = 16384
N_COLS = 256
N_OUT = 16
L = 16

NC = 2
NS = 16
NW = NC * NS
ROWS_PER_W = N_ROWS // NW
CHUNK = 128
NCHUNK = ROWS_PER_W // CHUNK


_mesh = plsc.VectorSubcoreMesh(core_axis_name="c", subcore_axis_name="s")


@functools.partial(
    pl.kernel,
    mesh=_mesh,
    out_type=jax.ShapeDtypeStruct((N_ROWS, N_OUT), jnp.float32),
    scratch_types=[
        pltpu.VMEM((2, CHUNK, N_COLS), jnp.float32),
        pltpu.VMEM((2, CHUNK, N_OUT), jnp.float32),
        pltpu.SemaphoreType.DMA,
        pltpu.SemaphoreType.DMA,
        pltpu.SemaphoreType.DMA,
        pltpu.SemaphoreType.DMA,
    ],
)
def _pool_sc(x_hbm, out_hbm, in_v, out_v, sem_in0, sem_in1, sem_out0, sem_out1):
    wid = lax.axis_index("s") * NC + lax.axis_index("c")
    base = wid * ROWS_PER_W
    sem_in = (sem_in0, sem_in1)
    sem_out = (sem_out0, sem_out1)

    def in_copy(ci):
        return pltpu.make_async_copy(
            x_hbm.at[pl.ds(base + ci * CHUNK, CHUNK)], in_v.at[ci % 2],
            sem_in[ci % 2])

    def out_copy(ci):
        return pltpu.make_async_copy(
            out_v.at[ci % 2], out_hbm.at[pl.ds(base + ci * CHUNK, CHUNK)],
            sem_out[ci % 2])

    in_copy(0).start()
    for ci in range(NCHUNK):
        buf = ci % 2
        if ci + 1 < NCHUNK:
            in_copy(ci + 1).start()
        if ci >= 2:
            out_copy(ci - 2).wait()
        in_copy(ci).wait()

        def row_body(r, _):
            v = in_v[buf, r, pl.ds(0, L)]
            for k in range(1, 16):
                v = jnp.maximum(v, in_v[buf, r, pl.ds(k * L, L)])
            out_v[buf, r, :] = v
            return 0

        lax.fori_loop(0, CHUNK, row_body, 0, unroll=4)
        out_copy(ci).start()

    out_copy(NCHUNK - 2).wait()
    out_copy(NCHUNK - 1).wait()


def kernel(x):
    return _pool_sc(x)


# speedup vs baseline: 4.0533x; 1.0287x over previous
import functools

import jax
import jax.numpy as jnp
from jax import lax
from jax.experimental import pallas as pl
from jax.experimental.pallas import tpu as pltpu
from jax.experimental.pallas import tpu_sc as plsc

N_ROWS = 16384
N_COLS = 256
N_OUT = 16
L = 16

NC = 2
NS = 16
NW = NC * NS
ROWS_PER_W = N_ROWS // NW
CHUNK = 128
NCHUNK = ROWS_PER_W // CHUNK


_mesh = plsc.VectorSubcoreMesh(core_axis_name="c", subcore_axis_name="s")


@functools.partial(
    pl.kernel,
    mesh=_mesh,
    out_type=jax.ShapeDtypeStruct((N_ROWS, N_OUT), jnp.float32),
    scratch_types=[
        pltpu.VMEM((2, CHUNK, N_COLS), jnp.float32),
        pltpu.VMEM((2, CHUNK, N_OUT), jnp.float32),
        pltpu.SemaphoreType.DMA,
        pltpu.SemaphoreType.DMA,
        pltpu.SemaphoreType.DMA,
        pltpu.SemaphoreType.DMA,
    ],
)
def _pool_sc(x_hbm, out_hbm, in_v, out_v, sem_in0, sem_in1, sem_out0, sem_out1):
    wid = lax.axis_index("s") * NC + lax.axis_index("c")
    base = wid * ROWS_PER_W
    sem_in = (sem_in0, sem_in1)
    sem_out = (sem_out0, sem_out1)

    def in_copy(ci):
        return pltpu.make_async_copy(
            x_hbm.at[pl.ds(base + ci * CHUNK, CHUNK)], in_v.at[ci % 2],
            sem_in[ci % 2])

    def out_copy(ci):
        return pltpu.make_async_copy(
            out_v.at[ci % 2], out_hbm.at[pl.ds(base + ci * CHUNK, CHUNK)],
            sem_out[ci % 2])

    in_copy(0).start()
    for ci in range(NCHUNK):
        buf = ci % 2
        if ci + 1 < NCHUNK:
            in_copy(ci + 1).start()
        if ci >= 2:
            out_copy(ci - 2).wait()
        in_copy(ci).wait()

        @plsc.parallel_loop(0, CHUNK, unroll=4)
        def row_body(r):
            vs = [in_v[buf, r, pl.ds(k * L, L)] for k in range(16)]
            while len(vs) > 1:
                vs = [jnp.maximum(vs[i], vs[i + 1])
                      for i in range(0, len(vs), 2)]
            out_v[buf, r, :] = vs[0]
        out_copy(ci).start()

    out_copy(NCHUNK - 2).wait()
    out_copy(NCHUNK - 1).wait()


def kernel(x):
    return _pool_sc(x)


# speedup vs baseline: 4.0846x; 1.0077x over previous
import functools

import jax
import jax.numpy as jnp
from jax import lax
from jax.experimental import pallas as pl
from jax.experimental.pallas import tpu as pltpu
from jax.experimental.pallas import tpu_sc as plsc

N_ROWS = 16384
N_COLS = 256
N_OUT = 16
L = 16

NC = 2
NS = 16
NW = NC * NS
ROWS_PER_W = N_ROWS // NW
CHUNK = 128
NCHUNK = ROWS_PER_W // CHUNK


_mesh = plsc.VectorSubcoreMesh(core_axis_name="c", subcore_axis_name="s")


@functools.partial(
    pl.kernel,
    mesh=_mesh,
    out_type=jax.ShapeDtypeStruct((N_ROWS, N_OUT), jnp.float32),
    compiler_params=pltpu.CompilerParams(use_tc_tiling_on_sc=True),
    scratch_types=[
        pltpu.VMEM((2, CHUNK, N_COLS), jnp.float32),
        pltpu.VMEM((2, CHUNK, N_OUT), jnp.float32),
        pltpu.SemaphoreType.DMA,
        pltpu.SemaphoreType.DMA,
        pltpu.SemaphoreType.DMA,
        pltpu.SemaphoreType.DMA,
    ],
)
def _pool_sc(x_hbm, out_hbm, in_v, out_v, sem_in0, sem_in1, sem_out0, sem_out1):
    wid = lax.axis_index("s") * NC + lax.axis_index("c")
    base = wid * ROWS_PER_W
    sem_in = (sem_in0, sem_in1)
    sem_out = (sem_out0, sem_out1)

    def in_copy(ci):
        return pltpu.make_async_copy(
            x_hbm.at[pl.ds(base + ci * CHUNK, CHUNK)], in_v.at[ci % 2],
            sem_in[ci % 2])

    def out_copy(ci):
        return pltpu.make_async_copy(
            out_v.at[ci % 2], out_hbm.at[pl.ds(base + ci * CHUNK, CHUNK)],
            sem_out[ci % 2])

    in_copy(0).start()
    for ci in range(NCHUNK):
        buf = ci % 2
        if ci + 1 < NCHUNK:
            in_copy(ci + 1).start()
        if ci >= 2:
            out_copy(ci - 2).wait()
        in_copy(ci).wait()

        @plsc.parallel_loop(0, CHUNK, unroll=4)
        def row_body(r):
            vs = [in_v[buf, r, pl.ds(k * L, L)] for k in range(16)]
            while len(vs) > 1:
                vs = [jnp.maximum(vs[i], vs[i + 1])
                      for i in range(0, len(vs), 2)]
            out_v[buf, r, :] = vs[0]
        out_copy(ci).start()

    out_copy(NCHUNK - 2).wait()
    out_copy(NCHUNK - 1).wait()


def kernel(x):
    return _pool_sc(x)
